# baseline (speedup 1.0000x reference)
import functools

import jax
import jax.numpy as jnp
from jax import lax
from jax.experimental import pallas as pl
from jax.experimental.pallas import tpu as pltpu
from jax.experimental.pallas import tpu_sc as plsc

_B = 16384 * 100
_D = 64
_NC = 2
_NS = 16
_NW = _NC * _NS
_BPW = _B // _NW
_G = 256
_K = 1
_CHUNK = _G * _K
_NCHUNK = _BPW // _CHUNK
_NBUF = 4

_mesh = plsc.VectorSubcoreMesh(core_axis_name="c", subcore_axis_name="s")


@functools.partial(
    pl.kernel,
    mesh=_mesh,
    out_type=jax.ShapeDtypeStruct((_B, _D), jnp.float32),
    scratch_types=[
        pltpu.VMEM((_NBUF, _K, _G), jnp.int32),
        pltpu.VMEM((_NBUF, _CHUNK, _D), jnp.float32),
    ] + [pltpu.SemaphoreType.DMA] * (2 * _NBUF),
    compiler_params=pltpu.CompilerParams(use_tc_tiling_on_sc=False),
)
def _sc_gather(table_hbm, idx_hbm, out_hbm, idx_v, rows_v, *sems):
    gsems = sems[:_NBUF]
    osems = sems[_NBUF:]
    wid = lax.axis_index("s") * _NC + lax.axis_index("c")
    row_base = wid * (_BPW // _G)
    out_base = wid * _BPW

    def fire_gathers(b, chunk):
        pltpu.sync_copy(idx_hbm.at[pl.ds(row_base + chunk * _K, _K)],
                        idx_v.at[b])
        for j in range(_K):
            pltpu.async_copy(
                table_hbm.at[idx_v.at[b].at[j]],
                rows_v.at[b].at[pl.ds(j * _G, _G)],
                gsems[b],
            )

    def drain_gathers(b):
        pltpu.make_async_copy(out_hbm.at[pl.ds(0, _CHUNK)], rows_v.at[b],
                              gsems[b]).wait()

    def fire_out(b, chunk):
        pltpu.async_copy(rows_v.at[b],
                         out_hbm.at[pl.ds(out_base + chunk * _CHUNK, _CHUNK)],
                         osems[b])

    def drain_out(b):
        pltpu.make_async_copy(out_hbm.at[pl.ds(0, _CHUNK)], rows_v.at[b],
                              osems[b]).wait()

    for b in range(_NBUF):
        fire_gathers(b, b)

    def body(io, carry):
        c0 = io * _NBUF
        for b in range(_NBUF):
            drain_gathers(b)
            fire_out(b, c0 + b)
        for b in range(_NBUF):
            drain_out(b)
            fire_gathers(b, c0 + _NBUF + b)
        return carry

    lax.fori_loop(0, _NCHUNK // _NBUF - 1, body, 0)

    last = _NCHUNK - _NBUF
    for b in range(_NBUF):
        drain_gathers(b)
        fire_out(b, last + b)
    for b in range(_NBUF):
        drain_out(b)


def kernel(data, idx):
    idx_flat = idx.reshape(_B // _G, _G).astype(jnp.int32)
    out = _sc_gather(data, idx_flat)
    out = out.reshape(idx.shape + (data.shape[1],))
    return out


# speedup vs baseline: 1.0029x; 1.0029x over previous
import functools

import jax
import jax.numpy as jnp
from jax import lax
from jax.experimental import pallas as pl
from jax.experimental.pallas import tpu as pltpu
from jax.experimental.pallas import tpu_sc as plsc

_B = 16384 * 100
_D = 64
_NC = 2
_NS = 16
_NW = _NC * _NS
_BPW = _B // _NW
_G = 800
_K = 1
_CHUNK = _G * _K
_NCHUNK = _BPW // _CHUNK
_NBUF = 2

_mesh = plsc.VectorSubcoreMesh(core_axis_name="c", subcore_axis_name="s")


@functools.partial(
    pl.kernel,
    mesh=_mesh,
    out_type=jax.ShapeDtypeStruct((_B, _D), jnp.float32),
    scratch_types=[
        pltpu.VMEM((_NBUF, _K, _G), jnp.int32),
        pltpu.VMEM((_NBUF, _CHUNK, _D), jnp.float32),
    ] + [pltpu.SemaphoreType.DMA] * (2 * _NBUF),
    compiler_params=pltpu.CompilerParams(use_tc_tiling_on_sc=False),
)
def _sc_gather(table_hbm, idx_hbm, out_hbm, idx_v, rows_v, *sems):
    gsems = sems[:_NBUF]
    osems = sems[_NBUF:]
    wid = lax.axis_index("s") * _NC + lax.axis_index("c")
    row_base = wid * (_BPW // _G)
    out_base = wid * _BPW

    def fire_gathers(b, chunk):
        pltpu.sync_copy(idx_hbm.at[pl.ds(row_base + chunk * _K, _K)],
                        idx_v.at[b])
        for j in range(_K):
            pltpu.async_copy(
                table_hbm.at[idx_v.at[b].at[j]],
                rows_v.at[b].at[pl.ds(j * _G, _G)],
                gsems[b],
            )

    def drain_gathers(b):
        pltpu.make_async_copy(out_hbm.at[pl.ds(0, _CHUNK)], rows_v.at[b],
                              gsems[b]).wait()

    def fire_out(b, chunk):
        pltpu.async_copy(rows_v.at[b],
                         out_hbm.at[pl.ds(out_base + chunk * _CHUNK, _CHUNK)],
                         osems[b])

    def drain_out(b):
        pltpu.make_async_copy(out_hbm.at[pl.ds(0, _CHUNK)], rows_v.at[b],
                              osems[b]).wait()

    for b in range(_NBUF):
        fire_gathers(b, b)

    def body(io, carry):
        c0 = io * _NBUF
        for b in range(_NBUF):
            drain_gathers(b)
            fire_out(b, c0 + b)
        for b in range(_NBUF):
            drain_out(b)
            fire_gathers(b, c0 + _NBUF + b)
        return carry

    lax.fori_loop(0, _NCHUNK // _NBUF - 1, body, 0)

    last = _NCHUNK - _NBUF
    for b in range(_NBUF):
        drain_gathers(b)
        fire_out(b, last + b)
    for b in range(_NBUF):
        drain_out(b)


def kernel(data, idx):
    idx_flat = idx.reshape(_B // _G, _G).astype(jnp.int32)
    out = _sc_gather(data, idx_flat)
    out = out.reshape(idx.shape + (data.shape[1],))
    return out
